# baseline (speedup 1.0000x reference)
import functools

import jax
import jax.numpy as jnp
from jax import lax
from jax.experimental import pallas as pl
from jax.experimental.pallas import tpu as pltpu
from jax.experimental.pallas import tpu_sc as plsc

N = 10000
E = 320000
G = 256
DF = 128
DE = 16
EO = 256
NO = 256
EX = 32
HID = 256
NEXP = 8

BE = 512
BN = 400
EP = 323584
NP = 10240

NC = 2
NS = 16
L = 16
C = 128


def _gather_rows(table, idx, d):
    B = idx.shape[0]
    P = B // (NC * NS)
    nch = P // C

    def body(table_hbm, idx_hbm, out_hbm, idx_v, rows_v, sem):
        wid = lax.axis_index("s") * NC + lax.axis_index("c")
        base = wid * P

        def step(i, carry):
            off = base + i * C
            pltpu.sync_copy(idx_hbm.at[pl.ds(off, C)], idx_v)
            pltpu.async_copy(table_hbm.at[idx_v], rows_v, sem).wait()
            pltpu.sync_copy(rows_v, out_hbm.at[pl.ds(off, C)])
            return carry

        lax.fori_loop(0, nch, step, 0)

    f = pl.kernel(
        body,
        out_type=jax.ShapeDtypeStruct((B, d), jnp.float32),
        mesh=plsc.VectorSubcoreMesh(core_axis_name="c", subcore_axis_name="s",
                                    num_cores=NC, num_subcores=NS),
        compiler_params=pltpu.CompilerParams(needs_layout_passes=False),
        scratch_types=[
            pltpu.VMEM((C,), jnp.int32),
            pltpu.VMEM((C, d), jnp.float32),
            pltpu.SemaphoreType.DMA,
        ],
    )
    return f(table, idx)


NW = NC * NS
ROWS_PT = NP // NW
TRASH = ROWS_PT
EBITS = 20
FLUSH = 128


def _compact_edges(seg):
    trash_entry = (TRASH << EBITS) | (EP - 1)
    nch = EP // FLUSH

    def body(seg_hbm, lists_hbm, counts_hbm, segbuf, listbuf, cbuf):
        c = lax.axis_index("c")
        s = lax.axis_index("s")
        t = s * NC + c
        base = t * ROWS_PT

        def step(i, carry):
            ptr, hbm_ptr = carry
            pltpu.sync_copy(seg_hbm.at[pl.ds(i * FLUSH, FLUSH)], segbuf)
            for j in range(FLUSH // L):
                v = segbuf[pl.ds(j * L, L)]
                ld = v - base
                ok = (ld >= 0) & (ld < ROWS_PT)
                eid = (i * FLUSH + j * L
                       + jax.lax.broadcasted_iota(jnp.int32, (L,), 0))
                packed = jnp.where(ok, eid | (ld << EBITS), trash_entry)
                oki = ok.astype(jnp.int32)
                pos = ptr + plsc.cumsum(oki) - oki
                plsc.store_scatter(listbuf, [pos], packed, mask=ok)
                ptr = ptr + jnp.sum(oki)

            def flush(args):
                p, hp = args
                off = pl.multiple_of(t * EP + hp, FLUSH)
                pltpu.sync_copy(listbuf.at[pl.ds(0, FLUSH)],
                                lists_hbm.at[pl.ds(off, FLUSH)])
                listbuf[pl.ds(0, L)] = listbuf[pl.ds(FLUSH, L)]
                return p - FLUSH, hp + FLUSH

            ptr, hbm_ptr = lax.cond(ptr >= FLUSH, flush,
                                    lambda a: a, (ptr, hbm_ptr))
            return ptr, hbm_ptr

        ptr, hbm_ptr = lax.fori_loop(0, nch, step,
                                     (jnp.int32(0), jnp.int32(0)))

        for k in range(FLUSH // L):
            listbuf[pl.ds(ptr + k * L, L)] = jnp.full((L,), trash_entry,
                                                      jnp.int32)
        @pl.when(ptr > 0)
        def _():
            off = pl.multiple_of(t * EP + hbm_ptr, FLUSH)
            pltpu.sync_copy(listbuf.at[pl.ds(0, FLUSH)],
                            lists_hbm.at[pl.ds(off, FLUSH)])

        total = hbm_ptr + ptr
        cbuf[pl.ds(0, L)] = jnp.full((L,), total, jnp.int32)
        pltpu.sync_copy(cbuf.at[pl.ds(0, L)],
                        counts_hbm.at[pl.ds(pl.multiple_of(t * L, 8), L)])

    f = pl.kernel(
        body,
        out_type=(jax.ShapeDtypeStruct((NW * EP,), jnp.int32),
                  jax.ShapeDtypeStruct((NW * L,), jnp.int32)),
        mesh=plsc.VectorSubcoreMesh(core_axis_name="c", subcore_axis_name="s",
                                    num_cores=NC, num_subcores=NS),
        compiler_params=pltpu.CompilerParams(needs_layout_passes=False),
        scratch_types=[
            pltpu.VMEM((FLUSH,), jnp.int32),
            pltpu.VMEM((2 * FLUSH + L,), jnp.int32),
            pltpu.VMEM((L,), jnp.int32),
        ],
    )
    return f(seg)


def _segment_sum(values, lists, counts, d):
    eid_mask = (1 << EBITS) - 1

    lane = None

    def body(val_hbm, lists_hbm, counts_hbm, out_hbm,
             acc, pbuf, eidx, rows_v, cbuf, sem):
        c = lax.axis_index("c")
        s = lax.axis_index("s")
        t = s * NC + c

        def zrow(r, carry):
            for j in range(d // L):
                acc[r, pl.ds(j * L, L)] = jnp.zeros((L,), jnp.float32)
            return carry

        lax.fori_loop(0, ROWS_PT + 1, zrow, 0)

        pltpu.sync_copy(counts_hbm.at[pl.ds(pl.multiple_of(t * L, 8), L)],
                        cbuf.at[pl.ds(0, L)])
        lane = jax.lax.broadcasted_iota(jnp.int32, (L,), 0)
        k_total = jnp.sum(jnp.where(lane == 0, cbuf[pl.ds(0, L)], 0))
        nch = (k_total + FLUSH - 1) // FLUSH

        def step(i, carry):
            off = pl.multiple_of(t * EP + i * FLUSH, FLUSH)
            pltpu.sync_copy(lists_hbm.at[pl.ds(off, FLUSH)], pbuf)
            for j in range(FLUSH // L):
                p = pbuf[pl.ds(j * L, L)]
                eidx[pl.ds(j * L, L)] = p & eid_mask
                pbuf[pl.ds(j * L, L)] = lax.shift_right_logical(p, EBITS)
            pltpu.async_copy(val_hbm.at[eidx], rows_v, sem).wait()

            def add_group(g, carry2):
                ldv = pbuf[pl.ds(pl.multiple_of(g * L, L), L)]
                for j2 in range(L):
                    ld = jnp.sum(jnp.where(lane == j2, ldv, 0))
                    row = g * L + j2
                    for q in range(d // L):
                        sl = pl.ds(q * L, L)
                        acc[ld, sl] = acc[ld, sl] + rows_v[row, sl]
                return carry2

            lax.fori_loop(0, FLUSH // L, add_group, 0)
            return carry

        lax.fori_loop(0, nch, step, 0)
        pltpu.sync_copy(acc.at[pl.ds(0, ROWS_PT)],
                        out_hbm.at[pl.ds(pl.multiple_of(t * ROWS_PT, 8),
                                         ROWS_PT)])

    f = pl.kernel(
        body,
        out_type=jax.ShapeDtypeStruct((NP, d), jnp.float32),
        mesh=plsc.VectorSubcoreMesh(core_axis_name="c", subcore_axis_name="s",
                                    num_cores=NC, num_subcores=NS),
        compiler_params=pltpu.CompilerParams(needs_layout_passes=False),
        scratch_types=[
            pltpu.VMEM((ROWS_PT + 1, d), jnp.float32),
            pltpu.VMEM((FLUSH,), jnp.int32),
            pltpu.VMEM((FLUSH,), jnp.int32),
            pltpu.VMEM((FLUSH, d), jnp.float32),
            pltpu.VMEM((L,), jnp.int32),
            pltpu.SemaphoreType.DMA,
        ],
    )
    return f(values, lists, counts)


def _mm_body(x_ref, w_ref, o_ref):
    o_ref[...] = jnp.dot(x_ref[...], w_ref[...],
                         preferred_element_type=jnp.float32)


def _mm(x, w, block_rows):
    m, k = x.shape
    n = w.shape[1]
    grid = m // block_rows
    return pl.pallas_call(
        _mm_body,
        grid=(grid,),
        in_specs=[
            pl.BlockSpec((block_rows, k), lambda i: (i, 0)),
            pl.BlockSpec((k, n), lambda i: (0, 0)),
        ],
        out_specs=pl.BlockSpec((block_rows, n), lambda i: (i, 0)),
        out_shape=jax.ShapeDtypeStruct((m, n), jnp.float32),
    )(x, w)


def _h0_body(xws_ref, ef_ref, web_ref, o_ref):
    o_ref[...] = jnp.maximum(
        xws_ref[...] + jnp.dot(ef_ref[...], web_ref[...],
                               preferred_element_type=jnp.float32), 0.0)


def _h0(xw_src, ef, web):
    grid = EP // BE
    return pl.pallas_call(
        _h0_body,
        grid=(grid,),
        in_specs=[
            pl.BlockSpec((BE, EO), lambda i: (i, 0)),
            pl.BlockSpec((BE, DE), lambda i: (i, 0)),
            pl.BlockSpec((DE, EO), lambda i: (0, 0)),
        ],
        out_specs=pl.BlockSpec((BE, EO), lambda i: (i, 0)),
        out_shape=jax.ShapeDtypeStruct((EP, EO), jnp.float32),
    )(xw_src, ef, web)


def _round_body(g1_ref, g2_ref, h0_ref, wu_ref, o_ref):
    m = g1_ref[...] - g2_ref[...]
    o_ref[...] = jnp.maximum(
        jnp.dot(m, wu_ref[...], preferred_element_type=jnp.float32)
        + h0_ref[...], 0.0)


def _round(g1, g2, h0, wu):
    grid = EP // BE
    return pl.pallas_call(
        _round_body,
        grid=(grid,),
        in_specs=[
            pl.BlockSpec((BE, EO), lambda i: (i, 0)),
            pl.BlockSpec((BE, EO), lambda i: (i, 0)),
            pl.BlockSpec((BE, EO), lambda i: (i, 0)),
            pl.BlockSpec((EO, EO), lambda i: (0, 0)),
        ],
        out_specs=pl.BlockSpec((BE, EO), lambda i: (i, 0)),
        out_shape=jax.ShapeDtypeStruct((EP, EO), jnp.float32),
    )(g1, g2, h0, wu)


def _node_body(x_ref, nm_ref, wa_ref, wb_ref, bn_ref, o_ref):
    acc = jnp.dot(x_ref[...], wa_ref[...], preferred_element_type=jnp.float32)
    acc += jnp.dot(nm_ref[...], wb_ref[...], preferred_element_type=jnp.float32)
    o_ref[...] = jnp.maximum(acc + bn_ref[...], 0.0)


def _node(x, node_m, wa, wb, bn):
    bn_rows = 512
    grid = NP // bn_rows
    return pl.pallas_call(
        _node_body,
        grid=(grid,),
        in_specs=[
            pl.BlockSpec((bn_rows, DF), lambda i: (i, 0)),
            pl.BlockSpec((bn_rows, EO), lambda i: (i, 0)),
            pl.BlockSpec((DF, NO), lambda i: (0, 0)),
            pl.BlockSpec((EO, NO), lambda i: (0, 0)),
            pl.BlockSpec((1, NO), lambda i: (0, 0)),
        ],
        out_specs=pl.BlockSpec((bn_rows, NO), lambda i: (i, 0)),
        out_shape=jax.ShapeDtypeStruct((NP, NO), jnp.float32),
    )(x, node_m, wa, wb, bn)


def _gseg_body(gid_ref, h_ref, o_ref):
    i = pl.program_id(0)

    @pl.when(i == 0)
    def _():
        o_ref[...] = jnp.zeros_like(o_ref)

    gid = gid_ref[0]
    groups = jax.lax.broadcasted_iota(jnp.int32, (G, 1), 0)
    onehot = jnp.where(gid == groups, 1.0, 0.0)
    o_ref[...] += jnp.dot(onehot, h_ref[...],
                          preferred_element_type=jnp.float32)


def _gseg(gid3, h):
    blk = 512
    grid = NP // blk
    return pl.pallas_call(
        _gseg_body,
        grid=(grid,),
        in_specs=[
            pl.BlockSpec((1, 1, blk), lambda i: (i, 0, 0)),
            pl.BlockSpec((blk, NO), lambda i: (i, 0)),
        ],
        out_specs=pl.BlockSpec((G, NO), lambda i: (0, 0)),
        out_shape=jax.ShapeDtypeStruct((G, NO), jnp.float32),
    )(gid3, h)


def _moe_body(c_ref, we1_ref, be1_ref, we2_ref, be2_ref, we3_ref, be3_ref,
              wg1_ref, bg1_ref, wg2_ref, bg2_ref, wg3_ref, bg3_ref, o_ref):
    hi = jax.lax.Precision.HIGHEST
    c = c_ref[...]
    g = jnp.maximum(jnp.dot(c, wg1_ref[...],
                            preferred_element_type=jnp.float32, precision=hi)
                    + bg1_ref[...], 0.0)
    g = jnp.maximum(jnp.dot(g, wg2_ref[...],
                            preferred_element_type=jnp.float32, precision=hi)
                    + bg2_ref[...], 0.0)
    logits = jnp.dot(g, wg3_ref[...],
                     preferred_element_type=jnp.float32, precision=hi) + bg3_ref[...]
    gate = jax.nn.softmax(logits, axis=1)
    acc = jnp.zeros((G, 1), jnp.float32)
    for e in range(NEXP):
        t = jnp.maximum(jnp.dot(c, we1_ref[e],
                                preferred_element_type=jnp.float32, precision=hi)
                        + be1_ref[e][None, :], 0.0)
        t = jnp.maximum(jnp.dot(t, we2_ref[e],
                                preferred_element_type=jnp.float32, precision=hi)
                        + be2_ref[e][None, :], 0.0)
        t = jnp.dot(t, we3_ref[e],
                    preferred_element_type=jnp.float32, precision=hi) + be3_ref[e][None, :]
        acc += t * gate[:, e:e + 1]
    o_ref[...] = acc


def _moe(c, p):
    in_feat = c.shape[1]
    full = lambda *s: pl.BlockSpec(s, lambda: tuple(0 for _ in s))
    return pl.pallas_call(
        _moe_body,
        in_specs=[
            full(G, in_feat),
            full(NEXP, in_feat, HID), full(NEXP, HID),
            full(NEXP, HID, HID), full(NEXP, HID),
            full(NEXP, HID, 1), full(NEXP, 1),
            full(in_feat, HID), full(1, HID),
            full(HID, HID), full(1, HID),
            full(HID, NEXP), full(1, NEXP),
        ],
        out_specs=full(G, 1),
        out_shape=jax.ShapeDtypeStruct((G, 1), jnp.float32),
    )(c, p['We1'], p['be1'], p['We2'], p['be2'], p['We3'], p['be3'],
      p['Wg1'], p['bg1'].reshape(1, HID), p['Wg2'], p['bg2'].reshape(1, HID),
      p['Wg3'], p['bg3'].reshape(1, NEXP))


def _mpnn(x, ef, src, dst, rev, we, wu, wn, bn, rounds):
    npad = EP - E
    src_p = jnp.pad(src, (0, npad)).astype(jnp.int32)
    dst_p = jnp.pad(dst, (0, npad), constant_values=NP).astype(jnp.int32)
    rev_p = jnp.pad(rev, (0, npad)).astype(jnp.int32)
    ef_p = jnp.pad(ef, ((0, npad), (0, 0)))
    x_p = jnp.pad(x, ((0, NP - N), (0, 0)))

    lists, counts = _compact_edges(dst_p)
    we_top, we_bot = we[:DF], we[DF:]
    xw = _mm(x, we_top, BN)
    h0 = _h0(_gather_rows(xw, src_p, EO), ef_p, we_bot)
    h = h0
    for _ in range(rounds):
        sum0 = _segment_sum(h, lists, counts, EO)
        h = _round(_gather_rows(sum0, src_p, EO),
                   _gather_rows(h, rev_p, EO), h0, wu)
    node_m = _segment_sum(h, lists, counts, EO)
    return _node(x_p, node_m, wn[:DF], wn[DF:], bn.reshape(1, NO))


def kernel(x_su, ef_su, src_su, dst_su, rev_su, gid_su,
           x_sv, ef_sv, src_sv, dst_sv, rev_sv, gid_sv,
           extra, params):
    p = params
    h_su = _mpnn(x_su, ef_su, src_su, dst_su, rev_su,
                 p['We_su'], p['Wu_su'], p['Wn_su'], p['bn_su'], 3)
    h_sv = _mpnn(x_sv, ef_sv, src_sv, dst_sv, rev_sv,
                 p['We_sv'], p['Wu_sv'], p['Wn_sv'], p['bn_sv'], 3)
    gid_su3 = jnp.pad(gid_su, (0, NP - N),
                      constant_values=G).astype(jnp.int32).reshape(-1, 1, 512)
    gid_sv3 = jnp.pad(gid_sv, (0, NP - N),
                      constant_values=G).astype(jnp.int32).reshape(-1, 1, 512)
    solute = _gseg(gid_su3, h_su)
    solvent = _gseg(gid_sv3, h_sv)
    combined = jnp.concatenate([solute, solvent, extra], axis=-1)
    return _moe(combined, p)


# speedup vs baseline: 1.0481x; 1.0481x over previous
import functools

import jax
import jax.numpy as jnp
from jax import lax
from jax.experimental import pallas as pl
from jax.experimental.pallas import tpu as pltpu
from jax.experimental.pallas import tpu_sc as plsc

N = 10000
E = 320000
G = 256
DF = 128
DE = 16
EO = 256
NO = 256
EX = 32
HID = 256
NEXP = 8

BE = 512
BN = 400
EP = 323584
NP = 10240

NC = 2
NS = 16
L = 16
C = 128


def _gather_rows(table, idx, d):
    B = idx.shape[0]
    P = B // (NC * NS)
    nch = P // C

    def body(table_hbm, idx_hbm, out_hbm,
             idx_v0, idx_v1, rows_v0, rows_v1, gsem, wsem0, wsem1):
        wid = lax.axis_index("s") * NC + lax.axis_index("c")
        base = wid * P
        idx_b = (idx_v0, idx_v1)
        rows_b = (rows_v0, rows_v1)
        wsems = (wsem0, wsem1)
        gathers = {}
        writes = {}

        def start(i):
            off = base + i * C
            pltpu.sync_copy(idx_hbm.at[pl.ds(off, C)], idx_b[i % 2])
            gathers[i] = pltpu.async_copy(table_hbm.at[idx_b[i % 2]],
                                          rows_b[i % 2], gsem)

        start(0)
        for i in range(nch):
            if i + 1 < nch:
                if i + 1 >= 2:
                    writes[i - 1].wait()
                start(i + 1)
            gathers[i].wait()
            writes[i] = pltpu.async_copy(
                rows_b[i % 2], out_hbm.at[pl.ds(base + i * C, C)],
                wsems[i % 2])
        if nch >= 2:
            writes[nch - 2].wait()
        writes[nch - 1].wait()

    f = pl.kernel(
        body,
        out_type=jax.ShapeDtypeStruct((B, d), jnp.float32),
        mesh=plsc.VectorSubcoreMesh(core_axis_name="c", subcore_axis_name="s",
                                    num_cores=NC, num_subcores=NS),
        compiler_params=pltpu.CompilerParams(needs_layout_passes=False),
        scratch_types=[
            pltpu.VMEM((C,), jnp.int32),
            pltpu.VMEM((C,), jnp.int32),
            pltpu.VMEM((C, d), jnp.float32),
            pltpu.VMEM((C, d), jnp.float32),
            pltpu.SemaphoreType.DMA,
            pltpu.SemaphoreType.DMA,
            pltpu.SemaphoreType.DMA,
        ],
    )
    return f(table, idx)


NW = NC * NS
ROWS_PT = NP // NW
TRASH = ROWS_PT
EBITS = 20
FLUSH = 128


def _compact_edges(seg):
    trash_entry = (TRASH << EBITS) | (EP - 1)
    nch = EP // FLUSH

    def body(seg_hbm, lists_hbm, counts_hbm, segbuf, listbuf, cbuf):
        c = lax.axis_index("c")
        s = lax.axis_index("s")
        t = s * NC + c
        base = t * ROWS_PT

        def step(i, carry):
            ptr, hbm_ptr = carry
            pltpu.sync_copy(seg_hbm.at[pl.ds(i * FLUSH, FLUSH)], segbuf)
            for j in range(FLUSH // L):
                v = segbuf[pl.ds(j * L, L)]
                ld = v - base
                ok = (ld >= 0) & (ld < ROWS_PT)
                eid = (i * FLUSH + j * L
                       + jax.lax.broadcasted_iota(jnp.int32, (L,), 0))
                packed = jnp.where(ok, eid | (ld << EBITS), trash_entry)
                oki = ok.astype(jnp.int32)
                pos = ptr + plsc.cumsum(oki) - oki
                plsc.store_scatter(listbuf, [pos], packed, mask=ok)
                ptr = ptr + jnp.sum(oki)

            def flush(args):
                p, hp = args
                off = pl.multiple_of(t * EP + hp, FLUSH)
                pltpu.sync_copy(listbuf.at[pl.ds(0, FLUSH)],
                                lists_hbm.at[pl.ds(off, FLUSH)])
                listbuf[pl.ds(0, L)] = listbuf[pl.ds(FLUSH, L)]
                return p - FLUSH, hp + FLUSH

            ptr, hbm_ptr = lax.cond(ptr >= FLUSH, flush,
                                    lambda a: a, (ptr, hbm_ptr))
            return ptr, hbm_ptr

        ptr, hbm_ptr = lax.fori_loop(0, nch, step,
                                     (jnp.int32(0), jnp.int32(0)))

        for k in range(FLUSH // L):
            listbuf[pl.ds(ptr + k * L, L)] = jnp.full((L,), trash_entry,
                                                      jnp.int32)
        @pl.when(ptr > 0)
        def _():
            off = pl.multiple_of(t * EP + hbm_ptr, FLUSH)
            pltpu.sync_copy(listbuf.at[pl.ds(0, FLUSH)],
                            lists_hbm.at[pl.ds(off, FLUSH)])

        total = hbm_ptr + ptr
        cbuf[pl.ds(0, L)] = jnp.full((L,), total, jnp.int32)
        pltpu.sync_copy(cbuf.at[pl.ds(0, L)],
                        counts_hbm.at[pl.ds(pl.multiple_of(t * L, 8), L)])

    f = pl.kernel(
        body,
        out_type=(jax.ShapeDtypeStruct((NW * EP,), jnp.int32),
                  jax.ShapeDtypeStruct((NW * L,), jnp.int32)),
        mesh=plsc.VectorSubcoreMesh(core_axis_name="c", subcore_axis_name="s",
                                    num_cores=NC, num_subcores=NS),
        compiler_params=pltpu.CompilerParams(needs_layout_passes=False),
        scratch_types=[
            pltpu.VMEM((FLUSH,), jnp.int32),
            pltpu.VMEM((2 * FLUSH + L,), jnp.int32),
            pltpu.VMEM((L,), jnp.int32),
        ],
    )
    return f(seg)


def _segment_sum(values, lists, counts, d):
    eid_mask = (1 << EBITS) - 1

    lane = None

    def body(val_hbm, lists_hbm, counts_hbm, out_hbm,
             acc, pbuf, eidx, rows_v, cbuf, sem):
        c = lax.axis_index("c")
        s = lax.axis_index("s")
        t = s * NC + c

        def zrow(r, carry):
            for j in range(d // L):
                acc[r, pl.ds(j * L, L)] = jnp.zeros((L,), jnp.float32)
            return carry

        lax.fori_loop(0, ROWS_PT + 1, zrow, 0)

        pltpu.sync_copy(counts_hbm.at[pl.ds(pl.multiple_of(t * L, 8), L)],
                        cbuf.at[pl.ds(0, L)])
        lane = jax.lax.broadcasted_iota(jnp.int32, (L,), 0)
        k_total = jnp.sum(jnp.where(lane == 0, cbuf[pl.ds(0, L)], 0))
        nch = (k_total + FLUSH - 1) // FLUSH

        def step(i, carry):
            off = pl.multiple_of(t * EP + i * FLUSH, FLUSH)
            pltpu.sync_copy(lists_hbm.at[pl.ds(off, FLUSH)], pbuf)
            for j in range(FLUSH // L):
                p = pbuf[pl.ds(j * L, L)]
                eidx[pl.ds(j * L, L)] = p & eid_mask
                pbuf[pl.ds(j * L, L)] = lax.shift_right_logical(p, EBITS)
            pltpu.async_copy(val_hbm.at[eidx], rows_v, sem).wait()

            def add_group(g, carry2):
                ldv = pbuf[pl.ds(pl.multiple_of(g * L, L), L)]
                for j2 in range(L):
                    ld = jnp.sum(jnp.where(lane == j2, ldv, 0))
                    row = g * L + j2
                    for q in range(d // L):
                        sl = pl.ds(q * L, L)
                        acc[ld, sl] = acc[ld, sl] + rows_v[row, sl]
                return carry2

            lax.fori_loop(0, FLUSH // L, add_group, 0)
            return carry

        lax.fori_loop(0, nch, step, 0)
        pltpu.sync_copy(acc.at[pl.ds(0, ROWS_PT)],
                        out_hbm.at[pl.ds(pl.multiple_of(t * ROWS_PT, 8),
                                         ROWS_PT)])

    f = pl.kernel(
        body,
        out_type=jax.ShapeDtypeStruct((NP, d), jnp.float32),
        mesh=plsc.VectorSubcoreMesh(core_axis_name="c", subcore_axis_name="s",
                                    num_cores=NC, num_subcores=NS),
        compiler_params=pltpu.CompilerParams(needs_layout_passes=False),
        scratch_types=[
            pltpu.VMEM((ROWS_PT + 1, d), jnp.float32),
            pltpu.VMEM((FLUSH,), jnp.int32),
            pltpu.VMEM((FLUSH,), jnp.int32),
            pltpu.VMEM((FLUSH, d), jnp.float32),
            pltpu.VMEM((L,), jnp.int32),
            pltpu.SemaphoreType.DMA,
        ],
    )
    return f(values, lists, counts)


def _mm_body(x_ref, w_ref, o_ref):
    o_ref[...] = jnp.dot(x_ref[...], w_ref[...],
                         preferred_element_type=jnp.float32)


def _mm(x, w, block_rows):
    m, k = x.shape
    n = w.shape[1]
    grid = m // block_rows
    return pl.pallas_call(
        _mm_body,
        grid=(grid,),
        in_specs=[
            pl.BlockSpec((block_rows, k), lambda i: (i, 0)),
            pl.BlockSpec((k, n), lambda i: (0, 0)),
        ],
        out_specs=pl.BlockSpec((block_rows, n), lambda i: (i, 0)),
        out_shape=jax.ShapeDtypeStruct((m, n), jnp.float32),
    )(x, w)


def _h0_body(xws_ref, ef_ref, web_ref, o_ref):
    o_ref[...] = jnp.maximum(
        xws_ref[...] + jnp.dot(ef_ref[...], web_ref[...],
                               preferred_element_type=jnp.float32), 0.0)


def _h0(xw_src, ef, web):
    grid = EP // BE
    return pl.pallas_call(
        _h0_body,
        grid=(grid,),
        in_specs=[
            pl.BlockSpec((BE, EO), lambda i: (i, 0)),
            pl.BlockSpec((BE, DE), lambda i: (i, 0)),
            pl.BlockSpec((DE, EO), lambda i: (0, 0)),
        ],
        out_specs=pl.BlockSpec((BE, EO), lambda i: (i, 0)),
        out_shape=jax.ShapeDtypeStruct((EP, EO), jnp.float32),
    )(xw_src, ef, web)


def _round_body(g1_ref, g2_ref, h0_ref, wu_ref, o_ref):
    m = g1_ref[...] - g2_ref[...]
    o_ref[...] = jnp.maximum(
        jnp.dot(m, wu_ref[...], preferred_element_type=jnp.float32)
        + h0_ref[...], 0.0)


def _round(g1, g2, h0, wu):
    grid = EP // BE
    return pl.pallas_call(
        _round_body,
        grid=(grid,),
        in_specs=[
            pl.BlockSpec((BE, EO), lambda i: (i, 0)),
            pl.BlockSpec((BE, EO), lambda i: (i, 0)),
            pl.BlockSpec((BE, EO), lambda i: (i, 0)),
            pl.BlockSpec((EO, EO), lambda i: (0, 0)),
        ],
        out_specs=pl.BlockSpec((BE, EO), lambda i: (i, 0)),
        out_shape=jax.ShapeDtypeStruct((EP, EO), jnp.float32),
    )(g1, g2, h0, wu)


def _node_body(x_ref, nm_ref, wa_ref, wb_ref, bn_ref, o_ref):
    acc = jnp.dot(x_ref[...], wa_ref[...], preferred_element_type=jnp.float32)
    acc += jnp.dot(nm_ref[...], wb_ref[...], preferred_element_type=jnp.float32)
    o_ref[...] = jnp.maximum(acc + bn_ref[...], 0.0)


def _node(x, node_m, wa, wb, bn):
    bn_rows = 512
    grid = NP // bn_rows
    return pl.pallas_call(
        _node_body,
        grid=(grid,),
        in_specs=[
            pl.BlockSpec((bn_rows, DF), lambda i: (i, 0)),
            pl.BlockSpec((bn_rows, EO), lambda i: (i, 0)),
            pl.BlockSpec((DF, NO), lambda i: (0, 0)),
            pl.BlockSpec((EO, NO), lambda i: (0, 0)),
            pl.BlockSpec((1, NO), lambda i: (0, 0)),
        ],
        out_specs=pl.BlockSpec((bn_rows, NO), lambda i: (i, 0)),
        out_shape=jax.ShapeDtypeStruct((NP, NO), jnp.float32),
    )(x, node_m, wa, wb, bn)


def _gseg_body(gid_ref, h_ref, o_ref):
    i = pl.program_id(0)

    @pl.when(i == 0)
    def _():
        o_ref[...] = jnp.zeros_like(o_ref)

    gid = gid_ref[0]
    groups = jax.lax.broadcasted_iota(jnp.int32, (G, 1), 0)
    onehot = jnp.where(gid == groups, 1.0, 0.0)
    o_ref[...] += jnp.dot(onehot, h_ref[...],
                          preferred_element_type=jnp.float32)


def _gseg(gid3, h):
    blk = 512
    grid = NP // blk
    return pl.pallas_call(
        _gseg_body,
        grid=(grid,),
        in_specs=[
            pl.BlockSpec((1, 1, blk), lambda i: (i, 0, 0)),
            pl.BlockSpec((blk, NO), lambda i: (i, 0)),
        ],
        out_specs=pl.BlockSpec((G, NO), lambda i: (0, 0)),
        out_shape=jax.ShapeDtypeStruct((G, NO), jnp.float32),
    )(gid3, h)


def _moe_body(c_ref, we1_ref, be1_ref, we2_ref, be2_ref, we3_ref, be3_ref,
              wg1_ref, bg1_ref, wg2_ref, bg2_ref, wg3_ref, bg3_ref, o_ref):
    hi = jax.lax.Precision.HIGHEST
    c = c_ref[...]
    g = jnp.maximum(jnp.dot(c, wg1_ref[...],
                            preferred_element_type=jnp.float32, precision=hi)
                    + bg1_ref[...], 0.0)
    g = jnp.maximum(jnp.dot(g, wg2_ref[...],
                            preferred_element_type=jnp.float32, precision=hi)
                    + bg2_ref[...], 0.0)
    logits = jnp.dot(g, wg3_ref[...],
                     preferred_element_type=jnp.float32, precision=hi) + bg3_ref[...]
    gate = jax.nn.softmax(logits, axis=1)
    acc = jnp.zeros((G, 1), jnp.float32)
    for e in range(NEXP):
        t = jnp.maximum(jnp.dot(c, we1_ref[e],
                                preferred_element_type=jnp.float32, precision=hi)
                        + be1_ref[e][None, :], 0.0)
        t = jnp.maximum(jnp.dot(t, we2_ref[e],
                                preferred_element_type=jnp.float32, precision=hi)
                        + be2_ref[e][None, :], 0.0)
        t = jnp.dot(t, we3_ref[e],
                    preferred_element_type=jnp.float32, precision=hi) + be3_ref[e][None, :]
        acc += t * gate[:, e:e + 1]
    o_ref[...] = acc


def _moe(c, p):
    in_feat = c.shape[1]
    full = lambda *s: pl.BlockSpec(s, lambda: tuple(0 for _ in s))
    return pl.pallas_call(
        _moe_body,
        in_specs=[
            full(G, in_feat),
            full(NEXP, in_feat, HID), full(NEXP, HID),
            full(NEXP, HID, HID), full(NEXP, HID),
            full(NEXP, HID, 1), full(NEXP, 1),
            full(in_feat, HID), full(1, HID),
            full(HID, HID), full(1, HID),
            full(HID, NEXP), full(1, NEXP),
        ],
        out_specs=full(G, 1),
        out_shape=jax.ShapeDtypeStruct((G, 1), jnp.float32),
    )(c, p['We1'], p['be1'], p['We2'], p['be2'], p['We3'], p['be3'],
      p['Wg1'], p['bg1'].reshape(1, HID), p['Wg2'], p['bg2'].reshape(1, HID),
      p['Wg3'], p['bg3'].reshape(1, NEXP))


def _mpnn(x, ef, src, dst, rev, we, wu, wn, bn, rounds):
    npad = EP - E
    src_p = jnp.pad(src, (0, npad)).astype(jnp.int32)
    dst_p = jnp.pad(dst, (0, npad), constant_values=NP).astype(jnp.int32)
    rev_p = jnp.pad(rev, (0, npad)).astype(jnp.int32)
    ef_p = jnp.pad(ef, ((0, npad), (0, 0)))
    x_p = jnp.pad(x, ((0, NP - N), (0, 0)))

    lists, counts = _compact_edges(dst_p)
    we_top, we_bot = we[:DF], we[DF:]
    xw = _mm(x, we_top, BN)
    h0 = _h0(_gather_rows(xw, src_p, EO), ef_p, we_bot)
    h = h0
    for _ in range(rounds):
        sum0 = _segment_sum(h, lists, counts, EO)
        h = _round(_gather_rows(sum0, src_p, EO),
                   _gather_rows(h, rev_p, EO), h0, wu)
    node_m = _segment_sum(h, lists, counts, EO)
    return _node(x_p, node_m, wn[:DF], wn[DF:], bn.reshape(1, NO))


def kernel(x_su, ef_su, src_su, dst_su, rev_su, gid_su,
           x_sv, ef_sv, src_sv, dst_sv, rev_sv, gid_sv,
           extra, params):
    p = params
    h_su = _mpnn(x_su, ef_su, src_su, dst_su, rev_su,
                 p['We_su'], p['Wu_su'], p['Wn_su'], p['bn_su'], 3)
    h_sv = _mpnn(x_sv, ef_sv, src_sv, dst_sv, rev_sv,
                 p['We_sv'], p['Wu_sv'], p['Wn_sv'], p['bn_sv'], 3)
    gid_su3 = jnp.pad(gid_su, (0, NP - N),
                      constant_values=G).astype(jnp.int32).reshape(-1, 1, 512)
    gid_sv3 = jnp.pad(gid_sv, (0, NP - N),
                      constant_values=G).astype(jnp.int32).reshape(-1, 1, 512)
    solute = _gseg(gid_su3, h_su)
    solvent = _gseg(gid_sv3, h_sv)
    combined = jnp.concatenate([solute, solvent, extra], axis=-1)
    return _moe(combined, p)
